# baseline (speedup 1.0000x reference)
import jax
import jax.numpy as jnp
from jax import lax
from jax.experimental import pallas as pl
from jax.experimental.pallas import tpu as pltpu
from jax.experimental.pallas import tpu_sc as plsc

B = 16384
D = 32
PK = 4
W = D * PK
NC = 2
NS = 16
NW = NC * NS
BPW = B // NW
L = 16
CH = 4
C = BPW // CH
EGC = C // L

_IDX_NAMES = ("aUE", "aUC", "nAUE", "nAUC", "aBHE", "aBTE", "aBC",
              "nABHE", "nABTE", "nABC", "tUCC", "tUPC", "tBCC", "tBPC",
              "uniqE", "uniqUC", "uniqBC")
_SLOT = {n: i for i, n in enumerate(_IDX_NAMES)}
NIDX = len(_IDX_NAMES)


def _body(idx_hbm, marg_hbm, ent_hbm, uc_hbm, bch_hbm, bct_hbm,
          o0, o1, o2, o3, o4, o5, o6, o7, o8, o9, o10, o11, o12,
          idx_v, row_v, bufA0, bufA1, bufB0, bufB1,
          v0, v1, v2, v3, v4, v5, v6, v7, v8, v9, v10, v11, v12, marg_v,
          sem_idx, semA, semB, sem_out):
    out_v = [v0, v1, v2, v3, v4, v5, v6, v7, v8, v9, v10, v11, v12]
    cid = lax.axis_index("c")
    sid = lax.axis_index("s")
    wid = sid * NC + cid
    base = wid * BPW

    h0 = pltpu.async_copy(idx_hbm.at[wid], idx_v, sem_idx)
    h1 = pltpu.async_copy(marg_hbm, marg_v, sem_idx)
    h0.wait()
    h1.wait()

    lane = lax.iota(jnp.int32, L)

    def mkrow(g, _):
        j = g // (CH * EGC)
        k = (g // EGC) % CH
        s = pl.ds((g % EGC) * L, L)
        row_v[j, k, s] = lax.shift_right_logical(idx_v[j, k, s], 2)
        return 0
    lax.fori_loop(0, NIDX * CH * EGC, mkrow, 0)

    tbl = {"E": ent_hbm, "UC": uc_hbm, "BCH": bch_hbm, "BCT": bct_hbm}

    passes = [
        ([("E", _SLOT["aUE"]), ("UC", _SLOT["aUC"])], "member", (0, "set")),
        ([("E", _SLOT["aBHE"]), ("BCH", _SLOT["aBC"])], "member", (1, "set")),
        ([("E", _SLOT["aBTE"]), ("BCT", _SLOT["aBC"])], "member", (1, "add")),
        ([("E", _SLOT["nAUE"]), ("UC", _SLOT["nAUC"])], "member", (2, "hinge")),
        ([("E", _SLOT["nABHE"]), ("BCH", _SLOT["nABC"])], "member", (3, "set")),
        ([("E", _SLOT["nABTE"]), ("BCT", _SLOT["nABC"])], "member", (3, "hinge_add")),
        ([("UC", _SLOT["tUCC"]), ("UC", _SLOT["tUPC"])], "hier", (4, 6, "u")),
        ([("BCH", _SLOT["tBCC"]), ("BCH", _SLOT["tBPC"])], "hier", (5, 7, "h")),
        ([("BCT", _SLOT["tBCC"]), ("BCT", _SLOT["tBPC"])], "hier", (5, 7, "t")),
        ([("E", _SLOT["uniqE"])], "norm", (8,)),
        ([("UC", _SLOT["uniqUC"])], "uniqc", (9, 11, "set")),
        ([("BCH", _SLOT["uniqBC"])], "uniqc", (10, 12, "set")),
        ([("BCT", _SLOT["uniqBC"])], "uniqc", (10, 12, "add")),
    ]

    stages = [(p, k) for p in range(len(passes)) for k in range(CH)]
    pairs = [(bufA0, bufA1), (bufB0, bufB1)]
    sems = [semA, semB]

    def issue(si):
        p, k = stages[si]
        gathers = passes[p][0]
        bufs = pairs[si % 2]
        sem = sems[si % 2]
        hs = []
        for (tk, j), buf in zip(gathers, bufs):
            hs.append(pltpu.async_copy(tbl[tk].at[row_v.at[j, k]], buf, sem))
        return hs

    zero = jnp.zeros((L,), jnp.float32)

    def dim_loop(e_ref, c_ref, brow, epos, cpos, mode):
        if mode == "member":
            def db(d, acc):
                ge = plsc.load_gather(e_ref, [brow, epos + d])
                gc = plsc.load_gather(c_ref, [brow, cpos + d])
                t = (1.0 - gc) * ge
                return acc + t * t
            return lax.fori_loop(0, D, db, zero, unroll=4)
        if mode == "hier":
            def db(d, carry):
                a, dc, dp = carry
                gc = plsc.load_gather(e_ref, [brow, epos + d])
                gp = plsc.load_gather(c_ref, [brow, cpos + d])
                t = gc * (1.0 - gp)
                return (a + t * t, dc + jnp.abs(gc), dp + jnp.abs(gp))
            return lax.fori_loop(0, D, db, (zero, zero, zero), unroll=4)
        if mode == "norm":
            def db(d, acc):
                ge = plsc.load_gather(e_ref, [brow, epos + d])
                return acc + ge * ge
            return lax.fori_loop(0, D, db, zero, unroll=4)
        def db(d, carry):
            a, n = carry
            gc = plsc.load_gather(e_ref, [brow, epos + d])
            t = gc * (1.0 - gc)
            return (a + t * t, n + jnp.abs(gc))
        return lax.fori_loop(0, D, db, (zero, zero), unroll=4)

    def compute(si):
        p, k = stages[si]
        gathers, kind, args = passes[p]
        b0, b1 = pairs[si % 2]
        mvec = marg_v[...]
        js = [j for (_, j) in gathers]

        def outer(le, _):
            sl = pl.ds(k * C + le * L, L)
            brow = le * L + lane
            i0 = idx_v[js[0], k, pl.ds(le * L, L)]
            pos0 = (i0 & 3) * D
            if len(js) > 1:
                i1 = idx_v[js[1], k, pl.ds(le * L, L)]
                pos1 = (i1 & 3) * D
            else:
                pos1 = None
            if kind == "member":
                oi, op = args
                s = dim_loop(b0, b1, brow, pos0, pos1, "member")
                if op == "set":
                    out_v[oi][sl] = s
                elif op == "add":
                    out_v[oi][sl] = out_v[oi][sl] + s
                elif op == "hinge":
                    out_v[oi][sl] = jnp.maximum(mvec - s, 0.0)
                else:
                    out_v[oi][sl] = jnp.maximum(
                        mvec - (out_v[oi][sl] + s), 0.0)
            elif kind == "hier":
                ai, ci, part = args
                a, dc, dp = dim_loop(b0, b1, brow, pos0, pos1, "hier")
                if part == "u":
                    out_v[ai][sl] = a
                    out_v[ci][sl] = jnp.maximum(dc + 1.0 - dp, 0.0)
                elif part == "h":
                    out_v[ai][sl] = a
                    out_v[ci][sl] = dc - dp
                else:
                    out_v[ai][sl] = out_v[ai][sl] + a
                    out_v[ci][sl] = jnp.maximum(
                        out_v[ci][sl] + dc - dp + 1.0, 0.0)
            elif kind == "norm":
                (oi,) = args
                s = dim_loop(b0, None, brow, pos0, None, "norm")
                t = s - 1.0
                out_v[oi][sl] = t * t
            else:
                ai, ci, op = args
                a, n = dim_loop(b0, None, brow, pos0, None, "uniqc")
                h = jnp.maximum(1.0 - n, 0.0)
                if op == "set":
                    out_v[ai][sl] = a
                    out_v[ci][sl] = h
                else:
                    out_v[ai][sl] = out_v[ai][sl] + a
                    out_v[ci][sl] = out_v[ci][sl] + h
            return 0

        lax.fori_loop(0, EGC, outer, 0)

    outs = [o0, o1, o2, o3, o4, o5, o6, o7, o8, o9, o10, o11, o12]
    done_after = {0: 0, 1: 2, 2: 3, 3: 5, 4: 6, 6: 6, 5: 8, 7: 8,
                  8: 9, 9: 10, 11: 10, 10: 12, 12: 12}

    out_handles = []
    hs = issue(0)
    for si in range(len(stages)):
        nxt = issue(si + 1) if si + 1 < len(stages) else []
        for h in hs:
            h.wait()
        compute(si)
        hs = nxt
        p, k = stages[si]
        if k == CH - 1:
            for oi, after in done_after.items():
                if after == p:
                    out_handles.append(pltpu.async_copy(
                        out_v[oi], outs[oi].at[pl.ds(base, BPW)], sem_out))
    for h in out_handles:
        h.wait()


def kernel(aUE, aUC, nAUE, nAUC, aBHE, aBTE, aBC, nABHE, nABTE, nABC,
           tUCC, tUPC, tBCC, tBPC, uniqE, uniqUC, uniqBC,
           rdHUC, rdTUC, rdBC, nRdHUC, nRdTUC, lossMargin, device,
           entityEmbed, uConceptEmbed, bConceptHEmbed, bConceptTEmbed):
    idx_arrays = (aUE, aUC, nAUE, nAUC, aBHE, aBTE, aBC, nABHE, nABTE,
                  nABC, tUCC, tUPC, tBCC, tBPC, uniqE, uniqUC, uniqBC)
    idx_all = jnp.stack(
        [a.reshape(NW, CH, C) for a in idx_arrays], axis=1)
    marg = jnp.broadcast_to(jnp.asarray(lossMargin, jnp.float32), (L,))

    ent_rm = entityEmbed.reshape(-1, W)
    uc_rm = uConceptEmbed.reshape(-1, W)
    bch_rm = bConceptHEmbed.reshape(-1, W)
    bct_rm = bConceptTEmbed.reshape(-1, W)

    mesh = plsc.VectorSubcoreMesh(core_axis_name="c", subcore_axis_name="s")
    out_type = tuple(jax.ShapeDtypeStruct((B,), jnp.float32)
                     for _ in range(13))
    f = pl.kernel(
        _body,
        out_type=out_type,
        mesh=mesh,
        compiler_params=pltpu.CompilerParams(
            use_tc_tiling_on_sc=False, needs_layout_passes=False),
        scratch_types=(
            [pltpu.VMEM((NIDX, CH, C), jnp.int32),
             pltpu.VMEM((NIDX, CH, C), jnp.int32)]
            + [pltpu.VMEM((C, W), jnp.float32) for _ in range(4)]
            + [pltpu.VMEM((BPW,), jnp.float32) for _ in range(13)]
            + [pltpu.VMEM((L,), jnp.float32)]
            + [pltpu.SemaphoreType.DMA for _ in range(4)]
        ),
    )
    return f(idx_all, marg, ent_rm, uc_rm, bch_rm, bct_rm)
